# baseline (speedup 1.0000x reference)
import functools

import jax
import jax.numpy as jnp
from jax import lax
from jax.experimental import pallas as pl
from jax.experimental.pallas import tpu as pltpu
from jax.experimental.pallas import tpu_sc as plsc

DIM2 = 512
N_CODES = 8192
N_TOK = 8192
BN = 256
RB = 1024
N_TILES = N_CODES // BN
D = 256


_HALF_TILES = N_TILES // 2


def _argmin_body(f_ref, fsq_ref, e_ref, esq_ref, out_ref,
                 minv_ref, mini_ref, c0v_ref, c0i_ref):
    n = pl.program_id(0)

    @pl.when((n == 0) | (n == _HALF_TILES))
    def _init():
        minv_ref[...] = jnp.full((N_TOK, 1), jnp.inf, jnp.float32)
        mini_ref[...] = jnp.zeros((N_TOK, 1), jnp.int32)

    e = e_ref[...].astype(jnp.bfloat16)
    esq = esq_ref[...]
    for mi in range(N_TOK // RB):
        rows = pl.ds(mi * RB, RB)
        f = f_ref[rows, :].astype(jnp.bfloat16)
        mm = jnp.dot(f, e, preferred_element_type=jnp.float32)
        d = (fsq_ref[rows, :] - mm) + esq
        rmin = jnp.min(d, axis=1, keepdims=True)
        cols = lax.broadcasted_iota(jnp.int32, d.shape, 1)
        ridx = jnp.min(jnp.where(d == rmin, cols, BN), axis=1, keepdims=True)
        ridx = ridx + n * BN
        cur_v = minv_ref[rows, :]
        cur_i = mini_ref[rows, :]
        better = rmin < cur_v
        minv_ref[rows, :] = jnp.where(better, rmin, cur_v)
        mini_ref[rows, :] = jnp.where(better, ridx, cur_i)

    @pl.when(n == _HALF_TILES - 1)
    def _save_chunk0():
        c0v_ref[...] = minv_ref[...]
        c0i_ref[...] = mini_ref[...]

    @pl.when(n == N_TILES - 1)
    def _emit():
        m0b = c0v_ref[...].astype(jnp.bfloat16).astype(jnp.float32)
        take1 = minv_ref[...] < m0b
        out_ref[...] = jnp.where(take1, mini_ref[...], c0i_ref[...])


def _fused_argmin(flatten2, fsq, embed, esq):
    return pl.pallas_call(
        _argmin_body,
        grid=(N_TILES,),
        in_specs=[
            pl.BlockSpec((N_TOK, DIM2), lambda n: (0, 0)),
            pl.BlockSpec((N_TOK, 1), lambda n: (0, 0)),
            pl.BlockSpec((DIM2, BN), lambda n: (0, n)),
            pl.BlockSpec((1, BN), lambda n: (0, n)),
        ],
        out_specs=pl.BlockSpec((N_TOK, 1), lambda n: (0, 0)),
        out_shape=jax.ShapeDtypeStruct((N_TOK, 1), jnp.int32),
        scratch_shapes=[
            pltpu.VMEM((N_TOK, 1), jnp.float32),
            pltpu.VMEM((N_TOK, 1), jnp.int32),
            pltpu.VMEM((N_TOK, 1), jnp.float32),
            pltpu.VMEM((N_TOK, 1), jnp.int32),
        ],
    )(flatten2, fsq, embed, esq)


_CHUNK = 128


@functools.lru_cache(maxsize=1)
def _build_sc_gather():
    mesh = plsc.VectorSubcoreMesh(core_axis_name="c", subcore_axis_name="s")

    @functools.partial(
        pl.kernel,
        out_type=[
            jax.ShapeDtypeStruct((N_TOK, D), jnp.float32),
            jax.ShapeDtypeStruct((N_TOK, D), jnp.float32),
        ],
        mesh=mesh,
        scratch_types=[
            pltpu.VMEM((2, _CHUNK), jnp.int32),
            pltpu.VMEM((_CHUNK, D), jnp.float32),
            pltpu.VMEM((_CHUNK, D), jnp.float32),
            pltpu.SemaphoreType.DMA,
            pltpu.SemaphoreType.DMA,
        ],
    )
    def _sc_gather(idx_hbm, hrt_hbm, lrt_hbm, out_hr, out_lr,
                   idx_v, rows_a, rows_b, sem_a, sem_b):
        wid = lax.axis_index("s") * 2 + lax.axis_index("c")
        base = wid * (2 * _CHUNK)
        pltpu.sync_copy(idx_hbm.at[pl.ds(wid * 2, 2)], idx_v)
        for k in range(2):
            cp_a = pltpu.async_copy(hrt_hbm.at[idx_v.at[k]], rows_a, sem_a)
            cp_b = pltpu.async_copy(lrt_hbm.at[idx_v.at[k]], rows_b, sem_b)
            cp_a.wait()
            pltpu.sync_copy(rows_a, out_hr.at[pl.ds(base + k * _CHUNK, _CHUNK)])
            cp_b.wait()
            pltpu.sync_copy(rows_b, out_lr.at[pl.ds(base + k * _CHUNK, _CHUNK)])

    return _sc_gather


_K3_RB = 1024
_K3_STEPS = N_TOK // _K3_RB
_INV_N = 1.0 / float(N_TOK * D)


def _st_body(qh_ref, qlr_ref, xh_ref, xlr_ref,
             oh_ref, olr_ref, sh_ref, slr_ref):
    i = pl.program_id(0)
    dh = qh_ref[...] - xh_ref[...]
    dl = qlr_ref[...] - xlr_ref[...]
    oh_ref[...] = xh_ref[...] + dh
    olr_ref[...] = xlr_ref[...] + dl
    sh = jnp.sum(dh * dh).reshape(1, 1)
    sl = jnp.sum(dl * dl).reshape(1, 1)
    acc_h = jnp.where(i == 0, sh, sh_ref[...] + sh)
    acc_l = jnp.where(i == 0, sl, slr_ref[...] + sl)
    last = i == _K3_STEPS - 1
    sh_ref[...] = jnp.where(last, acc_h * _INV_N, acc_h)
    slr_ref[...] = jnp.where(last, acc_l * _INV_N, acc_l)


def _straight_through(q_hr, q_lr, x_hr, x_lr):
    row_spec = pl.BlockSpec((_K3_RB, D), lambda i: (i, 0))
    one_spec = pl.BlockSpec((1, 1), lambda i: (0, 0))
    return pl.pallas_call(
        _st_body,
        grid=(_K3_STEPS,),
        in_specs=[row_spec] * 4,
        out_specs=[row_spec, row_spec, one_spec, one_spec],
        out_shape=[
            jax.ShapeDtypeStruct((N_TOK, D), jnp.float32),
            jax.ShapeDtypeStruct((N_TOK, D), jnp.float32),
            jax.ShapeDtypeStruct((1, 1), jnp.float32),
            jax.ShapeDtypeStruct((1, 1), jnp.float32),
        ],
    )(q_hr, q_lr, x_hr, x_lr)


def kernel(input_hr, input_lr, embed_lr, embed_hr):
    dim = input_hr.shape[-1]
    flatten_hr = input_hr.reshape(-1, dim)
    flatten_lr = input_lr.reshape(-1, dim)
    flatten = jnp.concatenate([flatten_lr, flatten_hr], axis=1)
    embed = jnp.concatenate([embed_lr, embed_hr], axis=0)
    fsq = (flatten ** 2).sum(1, keepdims=True)
    esq = (embed ** 2).sum(0, keepdims=True)

    ind2d = _fused_argmin(2.0 * flatten, fsq, embed, esq)

    idx_hbm = ind2d.reshape(N_TOK // _CHUNK, _CHUNK)
    q_hr, q_lr = _build_sc_gather()(idx_hbm, embed_hr.T, embed_lr.T)

    o_hr, o_lr, s_hr, s_lr = _straight_through(
        q_hr, q_lr, flatten_hr, flatten_lr)

    embed_ind = ind2d.reshape(input_hr.shape[:-1])
    quantize_hr = o_hr.reshape(input_hr.shape)
    quantize_lr = o_lr.reshape(input_lr.shape)
    diff_hr = s_hr.reshape(())
    diff_lr = s_lr.reshape(())
    return (quantize_hr, quantize_lr, diff_hr, diff_lr, embed_ind, embed_ind)
